# Initial kernel scaffold; baseline (speedup 1.0000x reference)
import jax
import jax.numpy as jnp
from jax.experimental import pallas as pl


def kernel(x, y, num_chars, num_labels):
    raise NotImplementedError("write your pallas kernel here")


# speedup vs baseline: 5.5787x; 5.5787x over previous
import jax
import jax.numpy as jnp
from jax.experimental import pallas as pl
from jax.experimental.pallas import tpu as pltpu

_NMAX = 2048
_W = _NMAX // 8
_XW = _NMAX + 8

_CP = getattr(pltpu, "CompilerParams", None) or getattr(pltpu, "TPUCompilerParams")


def _stats_kernel(x_ref, pred_ref, lse_ref):
    xb = x_ref[...]
    pred_ref[...] = jnp.argmax(xb, axis=1, keepdims=True).astype(jnp.int32)
    mx = jnp.max(xb, axis=1, keepdims=True)
    lse_ref[...] = mx + jnp.log(jnp.sum(jnp.exp(xb - mx), axis=1, keepdims=True))


def _seg_kernel(ns_ref, ms_ref, poff_ref, aw_ref, yw_ref, lw_ref, bw_ref,
                lab_ref, lsesum_ref, code_ref):
    n = ns_ref[pl.program_id(0)]
    m = ms_ref[pl.program_id(0)]
    poff = poff_ref[pl.program_id(0)]

    b = bw_ref[0]
    jpos = (jax.lax.broadcasted_iota(jnp.int32, (8, _W), 0) * _W
            + jax.lax.broadcasted_iota(jnp.int32, (8, _W), 1))
    row0 = jpos + 1
    big = jnp.int32(1 << 24)

    def flat_shift1(v, first):
        bcol = jnp.concatenate(
            [jnp.full((1, 1), first, jnp.int32), v[:-1, -1:]], axis=0)
        return jnp.concatenate([bcol, v[:, :-1]], axis=1)

    def flat_cummin(e):
        ps = e
        sft = 1
        while sft < _W:
            pad = jnp.full((8, sft), big, jnp.int32)
            ps = jnp.minimum(ps, jnp.concatenate([pad, ps[:, :-sft]], axis=1))
            sft *= 2
        rl = ps[:, -1:]
        ex = jnp.concatenate([jnp.full((1, 1), big, jnp.int32), rl[:-1]], axis=0)
        for sshift in (1, 2, 4):
            pad = jnp.full((sshift, 1), big, jnp.int32)
            ex = jnp.minimum(ex, jnp.concatenate([pad, ex[:-sshift]], axis=0))
        return jnp.minimum(ps, ex)

    def dp_row(i, prev):
        im1 = i - 1
        a_i = aw_ref[0, 0, im1]
        sub = jnp.where(b == a_i, 0, 1).astype(jnp.int32)
        prevm1 = flat_shift1(prev, im1)
        cand = jnp.minimum(prev + 1, prevm1 + sub)
        h = jnp.minimum(flat_cummin(cand - row0), i)
        row = row0 + h
        diag = row == prevm1 + sub
        up = row == prev + 1
        code = jnp.where(diag, row0 * 4,
                         jnp.where(up, row0 * 4 + 1, 0)).astype(jnp.int32)
        code_ref[pl.ds(im1, 1)] = code[None]
        return row

    jax.lax.fori_loop(1, n + 1, dp_row, row0)

    def clr(t, _):
        lab_ref[0, 0, t] = jnp.int32(-1)
        return 0

    jax.lax.fori_loop(0, _XW, clr, 0)

    def tb_cond(st):
        i, j, acc = st
        return (i > 0) & (j > 0)

    def tb_body(st):
        i, j, acc = st
        crow = code_ref[i - 1]
        cmax = jnp.max(jnp.where(row0 <= j, crow, 0))
        col = jax.lax.shift_right_logical(cmax, 2)
        found = cmax > 0
        isdiag = found & ((cmax & 3) == 0)

        @pl.when(isdiag)
        def _():
            lab_ref[0, 0, poff + i - 1] = yw_ref[0, 0, col - 1]

        acc = acc + jnp.where(isdiag, lw_ref[0, 0, i - 1], 0.0)
        i_new = jnp.where(found, i - 1, i)
        j_new = jnp.where(isdiag, col - 1, jnp.where(found, col, 0))
        return (i_new, j_new, acc)

    _, _, lacc = jax.lax.while_loop(
        tb_cond, tb_body, (n, m, jnp.float32(0.0)))
    lsesum_ref[0, 0, 0] = lacc


def _ce_kernel(ns_ref, ms_ref, xs_ref, lsesum_ref, lab_ref, x_ref,
               out_ref, acc_ref):
    s = pl.program_id(0)
    nseg = pl.num_programs(0)

    @pl.when(s == 0)
    def _():
        acc_ref[0] = 0.0
        acc_ref[1] = 0.0

    xstart = pl.multiple_of(xs_ref[s], 8)
    xb = x_ref[pl.ds(xstart, _XW), :]
    lab2 = lab_ref[0]
    lanes = jax.lax.broadcasted_iota(jnp.int32, (_XW, x_ref.shape[1]), 1)
    xdot = jnp.sum(jnp.where(lab2 == lanes, xb, 0.0))
    cnt = jnp.sum(jnp.where(lab2 >= 0, 1, 0))

    nonempty = (ns_ref[s] > 0) & (ms_ref[s] > 0)
    ce_sum = lsesum_ref[s, 0] - xdot
    seg_mean = ce_sum / jnp.maximum(cnt, 1).astype(jnp.float32)
    acc_ref[0] = acc_ref[0] + jnp.where(nonempty, seg_mean, 0.0)
    acc_ref[1] = acc_ref[1] + jnp.where(nonempty, 0.0, 1.0)

    @pl.when(s == nseg - 1)
    def _():
        out_ref[0, 0] = acc_ref[0] / (jnp.float32(nseg) - acc_ref[1])


def kernel(x, y, num_chars, num_labels):
    Lx, C = x.shape
    Ly = y.shape[0]
    S = num_chars.shape[0]

    y32 = y.astype(jnp.int32)
    nc = num_chars.astype(jnp.int32)
    nl = num_labels.astype(jnp.int32)

    BR = min(1024, Lx)
    pred2, lse2 = pl.pallas_call(
        _stats_kernel,
        grid=(Lx // BR,),
        in_specs=[pl.BlockSpec((BR, C), lambda i: (i, 0))],
        out_specs=[pl.BlockSpec((BR, 1), lambda i: (i, 0)),
                   pl.BlockSpec((BR, 1), lambda i: (i, 0))],
        out_shape=[jax.ShapeDtypeStruct((Lx, 1), jnp.int32),
                   jax.ShapeDtypeStruct((Lx, 1), jnp.float32)],
    )(x)
    pred = pred2[:, 0]
    lse = lse2[:, 0]

    pxs, pys, ns, ms = [], [], [], []
    px = jnp.int32(0)
    py = jnp.int32(0)
    for i in range(S):
        n_i = jnp.clip(jnp.minimum(nc[i], Lx - px), 0, _NMAX)
        m_i = jnp.clip(jnp.minimum(nl[i], Ly - py), 0, _NMAX)
        pxs.append(px)
        pys.append(py)
        ns.append(n_i)
        ms.append(m_i)
        ne = (n_i > 0) & (m_i > 0)
        px = px + jnp.where(ne, nc[i], 0)
        py = py + jnp.where(ne, nl[i], 0)
    pxs = jnp.stack(pxs)
    pys = jnp.stack(pys)
    ns = jnp.stack(ns)
    ms = jnp.stack(ms)

    xstarts = jnp.minimum((pxs // 8) * 8, Lx - _XW)
    poff = pxs - xstarts

    k2 = jnp.arange(_NMAX, dtype=jnp.int32)
    gx = jnp.clip(pxs[:, None] + k2[None, :], 0, Lx - 1)
    gy = jnp.clip(pys[:, None] + k2[None, :], 0, Ly - 1)
    aw = pred[gx]
    bw = y32[gy].reshape(S, 8, _W)
    yw = y32[gy]
    lw = lse[gx]

    lab, lsesum = pl.pallas_call(
        _seg_kernel,
        grid=(S,),
        in_specs=[
            pl.BlockSpec(memory_space=pltpu.SMEM),
            pl.BlockSpec(memory_space=pltpu.SMEM),
            pl.BlockSpec(memory_space=pltpu.SMEM),
            pl.BlockSpec((1, 1, _NMAX), lambda s: (s, 0, 0),
                         memory_space=pltpu.SMEM),
            pl.BlockSpec((1, 1, _NMAX), lambda s: (s, 0, 0),
                         memory_space=pltpu.SMEM),
            pl.BlockSpec((1, 1, _NMAX), lambda s: (s, 0, 0),
                         memory_space=pltpu.SMEM),
            pl.BlockSpec((1, 8, _W), lambda s: (s, 0, 0)),
        ],
        out_specs=[
            pl.BlockSpec((1, 1, _XW), lambda s: (s, 0, 0),
                         memory_space=pltpu.SMEM),
            pl.BlockSpec((1, 1, 1), lambda s: (s, 0, 0),
                         memory_space=pltpu.SMEM),
        ],
        out_shape=[jax.ShapeDtypeStruct((S, 1, _XW), jnp.int32),
                   jax.ShapeDtypeStruct((S, 1, 1), jnp.float32)],
        scratch_shapes=[pltpu.VMEM((_NMAX, 8, _W), jnp.int32)],
        compiler_params=_CP(vmem_limit_bytes=40 * 1024 * 1024),
    )(ns, ms, poff, aw[:, None, :], yw[:, None, :], lw[:, None, :], bw)

    out = pl.pallas_call(
        _ce_kernel,
        grid=(S,),
        in_specs=[
            pl.BlockSpec(memory_space=pltpu.SMEM),
            pl.BlockSpec(memory_space=pltpu.SMEM),
            pl.BlockSpec(memory_space=pltpu.SMEM),
            pl.BlockSpec(memory_space=pltpu.SMEM),
            pl.BlockSpec((1, _XW, 1), lambda s: (s, 0, 0)),
            pl.BlockSpec((Lx, C), lambda s: (0, 0)),
        ],
        out_specs=pl.BlockSpec(memory_space=pltpu.SMEM),
        out_shape=jax.ShapeDtypeStruct((1, 1), jnp.float32),
        scratch_shapes=[pltpu.SMEM((2,), jnp.float32)],
        compiler_params=_CP(vmem_limit_bytes=40 * 1024 * 1024),
    )(ns, ms, xstarts, lsesum.reshape(S, 1), lab.reshape(S, _XW, 1), x)
    return out[0, 0]


# speedup vs baseline: 12.1949x; 2.1860x over previous
import jax
import jax.numpy as jnp
from jax.experimental import pallas as pl
from jax.experimental.pallas import tpu as pltpu

_NMAX = 2048
_W = _NMAX // 8
_XW = _NMAX + 8

_CP = getattr(pltpu, "CompilerParams", None) or getattr(pltpu, "TPUCompilerParams")


def _stats_kernel(x_ref, pred_ref, lse_ref):
    xb = x_ref[...]
    pred_ref[...] = jnp.argmax(xb, axis=1, keepdims=True).astype(jnp.int32)
    mx = jnp.max(xb, axis=1, keepdims=True)
    lse_ref[...] = mx + jnp.log(jnp.sum(jnp.exp(xb - mx), axis=1, keepdims=True))


_DMAX = 2 * _NMAX


def _seg_kernel(ns_ref, ms_ref, poff_ref, yw_ref, lw_ref, aw_ref,
                lab_ref, lsesum_ref, rm_ref):
    n = ns_ref[pl.program_id(0)]
    m = ms_ref[pl.program_id(0)]
    poff = poff_ref[pl.program_id(0)]

    a_vec = aw_ref[0]
    jpos = (jax.lax.broadcasted_iota(jnp.int32, (8, _W), 0) * _W
            + jax.lax.broadcasted_iota(jnp.int32, (8, _W), 1))
    big = jnp.int32(1 << 24)

    def flat_shift1(v, first):
        bcol = jnp.concatenate(
            [jnp.full((1, 1), first, jnp.int32), v[:-1, -1:]], axis=0)
        return jnp.concatenate([bcol, v[:, :-1]], axis=1)

    def step(d, carry):
        vm1, shvm2, bb, r = carry
        shvm1 = flat_shift1(vm1, d - 1)
        bb2 = flat_shift1(bb, yw_ref[0, 0, jnp.clip(d - 2, 0, _NMAX - 1)])
        sub = jnp.where(a_vec == bb2, 0, 1).astype(jnp.int32)
        dcand = shvm2 + sub
        upc = shvm1 + 1
        comp = jnp.minimum(jnp.minimum(vm1 + 1, upc), dcand)
        vd = jnp.where(jpos == d - 1, d, comp)
        jc4 = ((d - 1) - jpos) * 4
        code = jnp.where(vd == dcand, jc4,
                         jnp.where(vd == upc, jc4 + 1, 0)).astype(jnp.int32)
        r2 = jnp.maximum(r, code)
        rm_ref[pl.ds(d - 2, 1)] = r2[None]
        return (vd, shvm1, bb2, r2)

    v1 = jnp.where(jpos == 0, 1, big).astype(jnp.int32)
    shv0 = jnp.where(jpos == 0, 0, big).astype(jnp.int32)
    zeros = jnp.zeros((8, _W), jnp.int32)
    jax.lax.fori_loop(2, n + m + 1, step, (v1, shv0, zeros, zeros))

    def clr(t, _):
        lab_ref[0, 0, t] = jnp.int32(-1)
        return 0

    jax.lax.fori_loop(0, _XW, clr, 0)

    def tb_cond(st):
        i, j, acc = st
        return (i > 0) & (j > 0)

    def tb_body(st):
        i, j, acc = st
        p = i - 1
        crow = rm_ref[i + j - 2]
        cmax = jnp.max(jnp.where(jpos == p, crow, 0))
        col = jax.lax.shift_right_logical(cmax, 2)
        found = cmax > 3
        isdiag = found & ((cmax & 3) == 0)
        lab = yw_ref[0, 0, jnp.clip(col - 1, 0, _NMAX - 1)]
        idx = jnp.where(isdiag, poff + p, _XW - 1)
        lab_ref[0, 0, idx] = jnp.where(isdiag, lab, jnp.int32(-1))
        acc = acc + jnp.where(isdiag, lw_ref[0, 0, p], 0.0)
        i2 = jnp.where(found, i - 1, i)
        j2 = jnp.where(isdiag, col - 1, jnp.where(found, col, 0))
        return (i2, j2, acc)

    _, _, lacc = jax.lax.while_loop(
        tb_cond, tb_body, (n, m, jnp.float32(0.0)))
    lsesum_ref[0, 0, 0] = lacc


def _ce_kernel(ns_ref, ms_ref, xs_ref, lsesum_ref, lab_ref, x_ref,
               out_ref, acc_ref):
    s = pl.program_id(0)
    nseg = pl.num_programs(0)

    @pl.when(s == 0)
    def _():
        acc_ref[0] = 0.0
        acc_ref[1] = 0.0

    xstart = pl.multiple_of(xs_ref[s], 8)
    xb = x_ref[pl.ds(xstart, _XW), :]
    lab2 = lab_ref[0]
    lanes = jax.lax.broadcasted_iota(jnp.int32, (_XW, x_ref.shape[1]), 1)
    xdot = jnp.sum(jnp.where(lab2 == lanes, xb, 0.0))
    cnt = jnp.sum(jnp.where(lab2 >= 0, 1, 0))

    nonempty = (ns_ref[s] > 0) & (ms_ref[s] > 0)
    ce_sum = lsesum_ref[s, 0] - xdot
    seg_mean = ce_sum / jnp.maximum(cnt, 1).astype(jnp.float32)
    acc_ref[0] = acc_ref[0] + jnp.where(nonempty, seg_mean, 0.0)
    acc_ref[1] = acc_ref[1] + jnp.where(nonempty, 0.0, 1.0)

    @pl.when(s == nseg - 1)
    def _():
        out_ref[0, 0] = acc_ref[0] / (jnp.float32(nseg) - acc_ref[1])


def kernel(x, y, num_chars, num_labels):
    Lx, C = x.shape
    Ly = y.shape[0]
    S = num_chars.shape[0]

    y32 = y.astype(jnp.int32)
    nc = num_chars.astype(jnp.int32)
    nl = num_labels.astype(jnp.int32)

    BR = min(1024, Lx)
    pred2, lse2 = pl.pallas_call(
        _stats_kernel,
        grid=(Lx // BR,),
        in_specs=[pl.BlockSpec((BR, C), lambda i: (i, 0))],
        out_specs=[pl.BlockSpec((BR, 1), lambda i: (i, 0)),
                   pl.BlockSpec((BR, 1), lambda i: (i, 0))],
        out_shape=[jax.ShapeDtypeStruct((Lx, 1), jnp.int32),
                   jax.ShapeDtypeStruct((Lx, 1), jnp.float32)],
    )(x)
    pred = pred2[:, 0]
    lse = lse2[:, 0]

    pxs, pys, ns, ms = [], [], [], []
    px = jnp.int32(0)
    py = jnp.int32(0)
    for i in range(S):
        n_i = jnp.clip(jnp.minimum(nc[i], Lx - px), 0, _NMAX)
        m_i = jnp.clip(jnp.minimum(nl[i], Ly - py), 0, _NMAX)
        pxs.append(px)
        pys.append(py)
        ns.append(n_i)
        ms.append(m_i)
        ne = (n_i > 0) & (m_i > 0)
        px = px + jnp.where(ne, nc[i], 0)
        py = py + jnp.where(ne, nl[i], 0)
    pxs = jnp.stack(pxs)
    pys = jnp.stack(pys)
    ns = jnp.stack(ns)
    ms = jnp.stack(ms)

    xstarts = jnp.minimum((pxs // 8) * 8, Lx - _XW)
    poff = pxs - xstarts

    k2 = jnp.arange(_NMAX, dtype=jnp.int32)
    gx = jnp.clip(pxs[:, None] + k2[None, :], 0, Lx - 1)
    gy = jnp.clip(pys[:, None] + k2[None, :], 0, Ly - 1)
    aw = pred[gx]
    yw = y32[gy]
    lw = lse[gx]

    lab, lsesum = pl.pallas_call(
        _seg_kernel,
        grid=(S,),
        in_specs=[
            pl.BlockSpec(memory_space=pltpu.SMEM),
            pl.BlockSpec(memory_space=pltpu.SMEM),
            pl.BlockSpec(memory_space=pltpu.SMEM),
            pl.BlockSpec((1, 1, _NMAX), lambda s: (s, 0, 0),
                         memory_space=pltpu.SMEM),
            pl.BlockSpec((1, 1, _NMAX), lambda s: (s, 0, 0),
                         memory_space=pltpu.SMEM),
            pl.BlockSpec((1, 8, _W), lambda s: (s, 0, 0)),
        ],
        out_specs=[
            pl.BlockSpec((1, 1, _XW), lambda s: (s, 0, 0),
                         memory_space=pltpu.SMEM),
            pl.BlockSpec((1, 1, 1), lambda s: (s, 0, 0),
                         memory_space=pltpu.SMEM),
        ],
        out_shape=[jax.ShapeDtypeStruct((S, 1, _XW), jnp.int32),
                   jax.ShapeDtypeStruct((S, 1, 1), jnp.float32)],
        scratch_shapes=[pltpu.VMEM((_DMAX, 8, _W), jnp.int32)],
        compiler_params=_CP(vmem_limit_bytes=48 * 1024 * 1024),
    )(ns, ms, poff, yw[:, None, :], lw[:, None, :], aw.reshape(S, 8, _W))

    out = pl.pallas_call(
        _ce_kernel,
        grid=(S,),
        in_specs=[
            pl.BlockSpec(memory_space=pltpu.SMEM),
            pl.BlockSpec(memory_space=pltpu.SMEM),
            pl.BlockSpec(memory_space=pltpu.SMEM),
            pl.BlockSpec(memory_space=pltpu.SMEM),
            pl.BlockSpec((1, _XW, 1), lambda s: (s, 0, 0)),
            pl.BlockSpec((Lx, C), lambda s: (0, 0)),
        ],
        out_specs=pl.BlockSpec(memory_space=pltpu.SMEM),
        out_shape=jax.ShapeDtypeStruct((1, 1), jnp.float32),
        scratch_shapes=[pltpu.SMEM((2,), jnp.float32)],
        compiler_params=_CP(vmem_limit_bytes=40 * 1024 * 1024),
    )(ns, ms, xstarts, lsesum.reshape(S, 1), lab.reshape(S, _XW, 1), x)
    return out[0, 0]


# speedup vs baseline: 16.3567x; 1.3413x over previous
import jax
import jax.numpy as jnp
from jax.experimental import pallas as pl
from jax.experimental.pallas import tpu as pltpu

_NMAX = 2048
_W = _NMAX // 8
_XW = _NMAX + 8

_CP = getattr(pltpu, "CompilerParams", None) or getattr(pltpu, "TPUCompilerParams")


def _stats_kernel(x_ref, pred_ref, lse_ref):
    xb = x_ref[...]
    pred_ref[...] = jnp.argmax(xb, axis=1, keepdims=True).astype(jnp.int32)
    mx = jnp.max(xb, axis=1, keepdims=True)
    lse_ref[...] = mx + jnp.log(jnp.sum(jnp.exp(xb - mx), axis=1, keepdims=True))


_DMAX = 2 * _NMAX


def _seg_kernel(ns_ref, ms_ref, poff_ref, yw_ref, lw_ref, aw_ref,
                lab_ref, lsesum_ref, rm_ref):
    s = pl.program_id(0)
    n0 = ns_ref[2 * s]
    m0 = ms_ref[2 * s]
    n1 = ns_ref[2 * s + 1]
    m1 = ms_ref[2 * s + 1]

    a0 = aw_ref[0, 0]
    a1 = aw_ref[0, 1]
    jpos = (jax.lax.broadcasted_iota(jnp.int32, (8, _W), 0) * _W
            + jax.lax.broadcasted_iota(jnp.int32, (8, _W), 1))
    jpos16 = (jax.lax.broadcasted_iota(jnp.int32, (16, _W), 0) * _W
              + jax.lax.broadcasted_iota(jnp.int32, (16, _W), 1))
    big = jnp.int32(1 << 24)

    def flat_shift1(v, first):
        bcol = jnp.concatenate(
            [jnp.full((1, 1), first, jnp.int32), v[:-1, -1:]], axis=0)
        return jnp.concatenate([bcol, v[:, :-1]], axis=1)

    def half_step(d, u, vm1, shvm2, bb, r, a_vec):
        shvm1 = flat_shift1(vm1, d - 1)
        bb2 = flat_shift1(bb, yw_ref[0, u, jnp.clip(d - 2, 0, _NMAX - 1)])
        sub = jnp.where(a_vec == bb2, 0, 1).astype(jnp.int32)
        dcand = shvm2 + sub
        upc = shvm1 + 1
        comp = jnp.minimum(jnp.minimum(vm1 + 1, upc), dcand)
        vd = jnp.where(jpos == d - 1, d, comp)
        jc4 = ((d - 1) - jpos) * 4
        code = jnp.where(vd == dcand, jc4,
                         jnp.where(vd == upc, jc4 + 1, 0)).astype(jnp.int32)
        r2 = jnp.maximum(r, code)
        return vd, shvm1, bb2, r2

    def step(d, carry):
        v0, sh0, bb0, r0, v1, sh1, bb1, r1 = carry
        v0, sh0, bb0, r0 = half_step(d, 0, v0, sh0, bb0, r0, a0)
        v1, sh1, bb1, r1 = half_step(d, 1, v1, sh1, bb1, r1, a1)
        rm_ref[pl.ds(d - 2, 1)] = (
            jnp.concatenate([r0, r1], axis=0).astype(jnp.int16)[None])
        return (v0, sh0, bb0, r0, v1, sh1, bb1, r1)

    vinit = jnp.where(jpos == 0, 1, big).astype(jnp.int32)
    shinit = jnp.where(jpos == 0, 0, big).astype(jnp.int32)
    zeros = jnp.zeros((8, _W), jnp.int32)
    dmax = jnp.maximum(n0 + m0, n1 + m1)
    jax.lax.fori_loop(2, dmax + 1, step,
                      (vinit, shinit, zeros, zeros,
                       vinit, shinit, zeros, zeros))

    def clr(t, _):
        lab_ref[0, 0, t] = jnp.int32(-1)
        lab_ref[0, 1, t] = jnp.int32(-1)
        return 0

    jax.lax.fori_loop(0, _XW, clr, 0)

    def tb_half(i, j, acc, u, active):
        p = i - 1
        crow = rm_ref[jnp.clip(i + j - 2, 0, _DMAX - 1)].astype(jnp.int32)
        cmax = jnp.max(jnp.where(jpos16 == 8 * _W * u + p, crow, 0))
        col = jax.lax.shift_right_logical(cmax, 2)
        found = active & (cmax > 3)
        isdiag = found & ((cmax & 3) == 0)
        lab = yw_ref[0, u, jnp.clip(col - 1, 0, _NMAX - 1)]
        idx = jnp.where(isdiag, poff_ref[2 * s + u] + p, _XW - 1)
        lab_ref[0, u, idx] = jnp.where(isdiag, lab, jnp.int32(-1))
        acc = acc + jnp.where(isdiag, lw_ref[0, u, p], 0.0)
        i2 = jnp.where(found, i - 1, i)
        j2 = jnp.where(isdiag, col - 1,
                       jnp.where(found, col, jnp.where(active, 0, j)))
        return i2, j2, acc

    def tb_cond(st):
        i0, j0, acc0, i1, j1, acc1 = st
        return ((i0 > 0) & (j0 > 0)) | ((i1 > 0) & (j1 > 0))

    def tb_body(st):
        i0, j0, acc0, i1, j1, acc1 = st
        i0, j0, acc0 = tb_half(i0, j0, acc0, 0, (i0 > 0) & (j0 > 0))
        i1, j1, acc1 = tb_half(i1, j1, acc1, 1, (i1 > 0) & (j1 > 0))
        return (i0, j0, acc0, i1, j1, acc1)

    _, _, acc0, _, _, acc1 = jax.lax.while_loop(
        tb_cond, tb_body,
        (n0, m0, jnp.float32(0.0), n1, m1, jnp.float32(0.0)))
    lsesum_ref[0, 0, 0] = acc0
    lsesum_ref[0, 1, 0] = acc1


def _ce_kernel(ns_ref, ms_ref, xs_ref, lsesum_ref, lab_ref, x_ref,
               out_ref, acc_ref):
    s = pl.program_id(0)
    nseg = pl.num_programs(0)

    @pl.when(s == 0)
    def _():
        acc_ref[0] = 0.0
        acc_ref[1] = 0.0

    xstart = pl.multiple_of(xs_ref[s], 8)
    xb = x_ref[pl.ds(xstart, _XW), :]
    lab2 = lab_ref[0]
    lanes = jax.lax.broadcasted_iota(jnp.int32, (_XW, x_ref.shape[1]), 1)
    xdot = jnp.sum(jnp.where(lab2 == lanes, xb, 0.0))
    cnt = jnp.sum(jnp.where(lab2 >= 0, 1, 0))

    nonempty = (ns_ref[s] > 0) & (ms_ref[s] > 0)
    ce_sum = lsesum_ref[s, 0] - xdot
    seg_mean = ce_sum / jnp.maximum(cnt, 1).astype(jnp.float32)
    acc_ref[0] = acc_ref[0] + jnp.where(nonempty, seg_mean, 0.0)
    acc_ref[1] = acc_ref[1] + jnp.where(nonempty, 0.0, 1.0)

    @pl.when(s == nseg - 1)
    def _():
        out_ref[0, 0] = acc_ref[0] / (jnp.float32(nseg) - acc_ref[1])


def kernel(x, y, num_chars, num_labels):
    Lx, C = x.shape
    Ly = y.shape[0]
    S = num_chars.shape[0]

    y32 = y.astype(jnp.int32)
    nc = num_chars.astype(jnp.int32)
    nl = num_labels.astype(jnp.int32)

    BR = min(1024, Lx)
    pred2, lse2 = pl.pallas_call(
        _stats_kernel,
        grid=(Lx // BR,),
        in_specs=[pl.BlockSpec((BR, C), lambda i: (i, 0))],
        out_specs=[pl.BlockSpec((BR, 1), lambda i: (i, 0)),
                   pl.BlockSpec((BR, 1), lambda i: (i, 0))],
        out_shape=[jax.ShapeDtypeStruct((Lx, 1), jnp.int32),
                   jax.ShapeDtypeStruct((Lx, 1), jnp.float32)],
    )(x)
    pred = pred2[:, 0]
    lse = lse2[:, 0]

    pxs, pys, ns, ms = [], [], [], []
    px = jnp.int32(0)
    py = jnp.int32(0)
    for i in range(S):
        n_i = jnp.clip(jnp.minimum(nc[i], Lx - px), 0, _NMAX)
        m_i = jnp.clip(jnp.minimum(nl[i], Ly - py), 0, _NMAX)
        pxs.append(px)
        pys.append(py)
        ns.append(n_i)
        ms.append(m_i)
        ne = (n_i > 0) & (m_i > 0)
        px = px + jnp.where(ne, nc[i], 0)
        py = py + jnp.where(ne, nl[i], 0)
    pxs = jnp.stack(pxs)
    pys = jnp.stack(pys)
    ns = jnp.stack(ns)
    ms = jnp.stack(ms)

    xstarts = jnp.minimum((pxs // 8) * 8, Lx - _XW)
    poff = pxs - xstarts

    k2 = jnp.arange(_NMAX, dtype=jnp.int32)
    gx = jnp.clip(pxs[:, None] + k2[None, :], 0, Lx - 1)
    gy = jnp.clip(pys[:, None] + k2[None, :], 0, Ly - 1)
    aw = pred[gx]
    yw = y32[gy]
    lw = lse[gx]

    lab, lsesum = pl.pallas_call(
        _seg_kernel,
        grid=(S // 2,),
        in_specs=[
            pl.BlockSpec(memory_space=pltpu.SMEM),
            pl.BlockSpec(memory_space=pltpu.SMEM),
            pl.BlockSpec(memory_space=pltpu.SMEM),
            pl.BlockSpec((1, 2, _NMAX), lambda s: (s, 0, 0),
                         memory_space=pltpu.SMEM),
            pl.BlockSpec((1, 2, _NMAX), lambda s: (s, 0, 0),
                         memory_space=pltpu.SMEM),
            pl.BlockSpec((1, 2, 8, _W), lambda s: (s, 0, 0, 0)),
        ],
        out_specs=[
            pl.BlockSpec((1, 2, _XW), lambda s: (s, 0, 0),
                         memory_space=pltpu.SMEM),
            pl.BlockSpec((1, 2, 1), lambda s: (s, 0, 0),
                         memory_space=pltpu.SMEM),
        ],
        out_shape=[jax.ShapeDtypeStruct((S // 2, 2, _XW), jnp.int32),
                   jax.ShapeDtypeStruct((S // 2, 2, 1), jnp.float32)],
        scratch_shapes=[pltpu.VMEM((_DMAX, 16, _W), jnp.int16)],
        compiler_params=_CP(vmem_limit_bytes=48 * 1024 * 1024),
    )(ns, ms, poff, yw.reshape(S // 2, 2, _NMAX), lw.reshape(S // 2, 2, _NMAX),
      aw.reshape(S // 2, 2, 8, _W))

    out = pl.pallas_call(
        _ce_kernel,
        grid=(S,),
        in_specs=[
            pl.BlockSpec(memory_space=pltpu.SMEM),
            pl.BlockSpec(memory_space=pltpu.SMEM),
            pl.BlockSpec(memory_space=pltpu.SMEM),
            pl.BlockSpec(memory_space=pltpu.SMEM),
            pl.BlockSpec((1, _XW, 1), lambda s: (s, 0, 0)),
            pl.BlockSpec((Lx, C), lambda s: (0, 0)),
        ],
        out_specs=pl.BlockSpec(memory_space=pltpu.SMEM),
        out_shape=jax.ShapeDtypeStruct((1, 1), jnp.float32),
        scratch_shapes=[pltpu.SMEM((2,), jnp.float32)],
        compiler_params=_CP(vmem_limit_bytes=40 * 1024 * 1024),
    )(ns, ms, xstarts, lsesum.reshape(S, 1), lab.reshape(S, _XW, 1), x)
    return out[0, 0]


# speedup vs baseline: 18.8810x; 1.1543x over previous
import jax
import jax.numpy as jnp
from jax.experimental import pallas as pl
from jax.experimental.pallas import tpu as pltpu

_NMAX = 2048
_W = _NMAX // 8
_XW = _NMAX + 8

_CP = getattr(pltpu, "CompilerParams", None) or getattr(pltpu, "TPUCompilerParams")


def _stats_kernel(x_ref, pred_ref, lse_ref):
    xb = x_ref[...]
    pred_ref[...] = jnp.argmax(xb, axis=1, keepdims=True).astype(jnp.int32)
    mx = jnp.max(xb, axis=1, keepdims=True)
    lse_ref[...] = mx + jnp.log(jnp.sum(jnp.exp(xb - mx), axis=1, keepdims=True))


_DMAX = 2 * _NMAX


def _seg_kernel(ns_ref, ms_ref, poff_ref, yw_ref, lw_ref, aw_ref,
                lab_ref, lsesum_ref, rm_ref):
    s = pl.program_id(0)
    n0 = ns_ref[2 * s]
    m0 = ms_ref[2 * s]
    n1 = ns_ref[2 * s + 1]
    m1 = ms_ref[2 * s + 1]

    a0 = aw_ref[0, 0]
    a1 = aw_ref[0, 1]
    jpos = (jax.lax.broadcasted_iota(jnp.int32, (8, _W), 0) * _W
            + jax.lax.broadcasted_iota(jnp.int32, (8, _W), 1))
    jpos16 = (jax.lax.broadcasted_iota(jnp.int32, (16, _W), 0) * _W
              + jax.lax.broadcasted_iota(jnp.int32, (16, _W), 1))
    big = jnp.int32(1 << 24)

    def flat_shiftw(v, w, fills):
        frow = jnp.concatenate(
            [jnp.full((1, 1), f, jnp.int32) for f in fills], axis=1)
        left = jnp.concatenate([frow, v[:-1, -w:]], axis=0)
        return jnp.concatenate([left, v[:, :-w]], axis=1)

    a0s = flat_shiftw(a0, 1, [0])
    a1s = flat_shiftw(a1, 1, [0])

    def half_step2(d, u, A, B, bb, r, a_vec, a_sh):
        yb0 = yw_ref[0, u, jnp.clip(d - 2, 0, _NMAX - 1)]
        yb1 = yw_ref[0, u, jnp.clip(d - 1, 0, _NMAX - 1)]
        A1 = flat_shiftw(A, 1, [d - 1])
        A2 = flat_shiftw(A, 2, [big, d - 1])
        B1 = flat_shiftw(B, 1, [big])
        bb1 = flat_shiftw(bb, 1, [yb0])
        bb2 = flat_shiftw(bb, 2, [yb1, yb0])
        sub0 = jnp.where(a_vec == bb1, 0, 1).astype(jnp.int32)
        sub1 = jnp.where(a_vec == bb2, 0, 1).astype(jnp.int32)
        subs = jnp.where(a_sh == bb2, 0, 1).astype(jnp.int32)
        dc0 = B + sub0
        up0 = A1 + 1
        vd = jnp.where(jpos == d - 1, d,
                       jnp.minimum(jnp.minimum(A + 1, up0), dc0))
        jc4 = ((d - 1) - jpos) * 4
        code0 = jnp.where(vd == dc0, jc4, jnp.where(vd == up0, jc4 + 1, 0))
        ra = jnp.maximum(r, code0)
        shvd = jnp.where(jpos == d, d,
                         jnp.minimum(jnp.minimum(A1 + 1, A2 + 1), B1 + subs))
        dc1 = A1 + sub1
        up1 = shvd + 1
        vd1 = jnp.where(jpos == d, d + 1,
                        jnp.minimum(jnp.minimum(vd + 1, up1), dc1))
        jc4b = (d - jpos) * 4
        code1 = jnp.where(vd1 == dc1, jc4b, jnp.where(vd1 == up1, jc4b + 1, 0))
        rb = jnp.maximum(ra, code1)
        return vd1, shvd, bb2, ra, rb

    def step(t, carry):
        v0, sh0, bb0, r0, v1, sh1, bb1, r1 = carry
        d = 2 + 2 * t
        v0, sh0, bb0, r0a, r0 = half_step2(d, 0, v0, sh0, bb0, r0, a0, a0s)
        v1, sh1, bb1, r1a, r1 = half_step2(d, 1, v1, sh1, bb1, r1, a1, a1s)
        rowa = jnp.concatenate([r0a, r1a], axis=0).astype(jnp.int16)
        rowb = jnp.concatenate([r0, r1], axis=0).astype(jnp.int16)
        rm_ref[pl.ds(d - 2, 2)] = jnp.concatenate(
            [rowa[None], rowb[None]], axis=0)
        return (v0, sh0, bb0, r0, v1, sh1, bb1, r1)

    vinit = jnp.where(jpos == 0, 1, big).astype(jnp.int32)
    shinit = jnp.where(jpos == 0, 0, big).astype(jnp.int32)
    zeros = jnp.zeros((8, _W), jnp.int32)
    dmax = jnp.maximum(n0 + m0, n1 + m1)
    jax.lax.fori_loop(0, dmax // 2, step,
                      (vinit, shinit, zeros, zeros,
                       vinit, shinit, zeros, zeros))

    def clr(t, _):
        for q in range(8):
            lab_ref[0, 0, t * 8 + q] = jnp.int32(-1)
            lab_ref[0, 1, t * 8 + q] = jnp.int32(-1)
        return 0

    jax.lax.fori_loop(0, _XW // 8, clr, 0)

    def tb_half(i, j, acc, u, active):
        p = i - 1
        crow = rm_ref[jnp.clip(i + j - 2, 0, _DMAX - 1)][
            8 * u:8 * u + 8].astype(jnp.int32)
        cmax = jnp.max(jnp.where(jpos == p, crow, 0))
        col = jax.lax.shift_right_logical(cmax, 2)
        found = active & (cmax > 3)
        isdiag = found & ((cmax & 3) == 0)
        lab = yw_ref[0, u, jnp.clip(col - 1, 0, _NMAX - 1)]
        idx = jnp.where(isdiag, poff_ref[2 * s + u] + p, _XW - 1)
        lab_ref[0, u, idx] = jnp.where(isdiag, lab, jnp.int32(-1))
        acc = acc + jnp.where(isdiag, lw_ref[0, u, p], 0.0)
        i2 = jnp.where(found, i - 1, i)
        j2 = jnp.where(isdiag, col - 1,
                       jnp.where(found, col, jnp.where(active, 0, j)))
        return i2, j2, acc

    def tb_cond(st):
        i0, j0, acc0, i1, j1, acc1 = st
        return ((i0 > 0) & (j0 > 0)) | ((i1 > 0) & (j1 > 0))

    def tb_body(st):
        i0, j0, acc0, i1, j1, acc1 = st
        i0, j0, acc0 = tb_half(i0, j0, acc0, 0, (i0 > 0) & (j0 > 0))
        i1, j1, acc1 = tb_half(i1, j1, acc1, 1, (i1 > 0) & (j1 > 0))
        return (i0, j0, acc0, i1, j1, acc1)

    _, _, acc0, _, _, acc1 = jax.lax.while_loop(
        tb_cond, tb_body,
        (n0, m0, jnp.float32(0.0), n1, m1, jnp.float32(0.0)))
    lsesum_ref[0, 0, 0] = acc0
    lsesum_ref[0, 1, 0] = acc1


def _ce_kernel(ns_ref, ms_ref, xs_ref, lsesum_ref, lab_ref, x_ref,
               out_ref, acc_ref):
    s = pl.program_id(0)
    nseg = pl.num_programs(0)

    @pl.when(s == 0)
    def _():
        acc_ref[0] = 0.0
        acc_ref[1] = 0.0

    xstart = pl.multiple_of(xs_ref[s], 8)
    xb = x_ref[pl.ds(xstart, _XW), :]
    lab2 = lab_ref[0]
    lanes = jax.lax.broadcasted_iota(jnp.int32, (_XW, x_ref.shape[1]), 1)
    xdot = jnp.sum(jnp.where(lab2 == lanes, xb, 0.0))
    cnt = jnp.sum(jnp.where(lab2 >= 0, 1, 0))

    nonempty = (ns_ref[s] > 0) & (ms_ref[s] > 0)
    ce_sum = lsesum_ref[s, 0] - xdot
    seg_mean = ce_sum / jnp.maximum(cnt, 1).astype(jnp.float32)
    acc_ref[0] = acc_ref[0] + jnp.where(nonempty, seg_mean, 0.0)
    acc_ref[1] = acc_ref[1] + jnp.where(nonempty, 0.0, 1.0)

    @pl.when(s == nseg - 1)
    def _():
        out_ref[0, 0] = acc_ref[0] / (jnp.float32(nseg) - acc_ref[1])


def kernel(x, y, num_chars, num_labels):
    Lx, C = x.shape
    Ly = y.shape[0]
    S = num_chars.shape[0]

    y32 = y.astype(jnp.int32)
    nc = num_chars.astype(jnp.int32)
    nl = num_labels.astype(jnp.int32)

    BR = min(1024, Lx)
    pred2, lse2 = pl.pallas_call(
        _stats_kernel,
        grid=(Lx // BR,),
        in_specs=[pl.BlockSpec((BR, C), lambda i: (i, 0))],
        out_specs=[pl.BlockSpec((BR, 1), lambda i: (i, 0)),
                   pl.BlockSpec((BR, 1), lambda i: (i, 0))],
        out_shape=[jax.ShapeDtypeStruct((Lx, 1), jnp.int32),
                   jax.ShapeDtypeStruct((Lx, 1), jnp.float32)],
    )(x)
    pred = pred2[:, 0]
    lse = lse2[:, 0]

    pxs, pys, ns, ms = [], [], [], []
    px = jnp.int32(0)
    py = jnp.int32(0)
    for i in range(S):
        n_i = jnp.clip(jnp.minimum(nc[i], Lx - px), 0, _NMAX)
        m_i = jnp.clip(jnp.minimum(nl[i], Ly - py), 0, _NMAX)
        pxs.append(px)
        pys.append(py)
        ns.append(n_i)
        ms.append(m_i)
        ne = (n_i > 0) & (m_i > 0)
        px = px + jnp.where(ne, nc[i], 0)
        py = py + jnp.where(ne, nl[i], 0)
    pxs = jnp.stack(pxs)
    pys = jnp.stack(pys)
    ns = jnp.stack(ns)
    ms = jnp.stack(ms)

    xstarts = jnp.minimum((pxs // 8) * 8, Lx - _XW)
    poff = pxs - xstarts

    k2 = jnp.arange(_NMAX, dtype=jnp.int32)
    gx = jnp.clip(pxs[:, None] + k2[None, :], 0, Lx - 1)
    gy = jnp.clip(pys[:, None] + k2[None, :], 0, Ly - 1)
    aw = pred[gx]
    yw = y32[gy]
    lw = lse[gx]

    lab, lsesum = pl.pallas_call(
        _seg_kernel,
        grid=(S // 2,),
        in_specs=[
            pl.BlockSpec(memory_space=pltpu.SMEM),
            pl.BlockSpec(memory_space=pltpu.SMEM),
            pl.BlockSpec(memory_space=pltpu.SMEM),
            pl.BlockSpec((1, 2, _NMAX), lambda s: (s, 0, 0),
                         memory_space=pltpu.SMEM),
            pl.BlockSpec((1, 2, _NMAX), lambda s: (s, 0, 0),
                         memory_space=pltpu.SMEM),
            pl.BlockSpec((1, 2, 8, _W), lambda s: (s, 0, 0, 0)),
        ],
        out_specs=[
            pl.BlockSpec((1, 2, _XW), lambda s: (s, 0, 0),
                         memory_space=pltpu.SMEM),
            pl.BlockSpec((1, 2, 1), lambda s: (s, 0, 0),
                         memory_space=pltpu.SMEM),
        ],
        out_shape=[jax.ShapeDtypeStruct((S // 2, 2, _XW), jnp.int32),
                   jax.ShapeDtypeStruct((S // 2, 2, 1), jnp.float32)],
        scratch_shapes=[pltpu.VMEM((_DMAX, 16, _W), jnp.int16)],
        compiler_params=_CP(vmem_limit_bytes=48 * 1024 * 1024),
    )(ns, ms, poff, yw.reshape(S // 2, 2, _NMAX), lw.reshape(S // 2, 2, _NMAX),
      aw.reshape(S // 2, 2, 8, _W))

    out = pl.pallas_call(
        _ce_kernel,
        grid=(S,),
        in_specs=[
            pl.BlockSpec(memory_space=pltpu.SMEM),
            pl.BlockSpec(memory_space=pltpu.SMEM),
            pl.BlockSpec(memory_space=pltpu.SMEM),
            pl.BlockSpec(memory_space=pltpu.SMEM),
            pl.BlockSpec((1, _XW, 1), lambda s: (s, 0, 0)),
            pl.BlockSpec((Lx, C), lambda s: (0, 0)),
        ],
        out_specs=pl.BlockSpec(memory_space=pltpu.SMEM),
        out_shape=jax.ShapeDtypeStruct((1, 1), jnp.float32),
        scratch_shapes=[pltpu.SMEM((2,), jnp.float32)],
        compiler_params=_CP(vmem_limit_bytes=40 * 1024 * 1024),
    )(ns, ms, xstarts, lsesum.reshape(S, 1), lab.reshape(S, _XW, 1), x)
    return out[0, 0]


# speedup vs baseline: 22.5737x; 1.1956x over previous
import jax
import jax.numpy as jnp
from jax.experimental import pallas as pl
from jax.experimental.pallas import tpu as pltpu

_NMAX = 2048
_W = _NMAX // 8
_XW = _NMAX + 8

_CP = getattr(pltpu, "CompilerParams", None) or getattr(pltpu, "TPUCompilerParams")


def _stats_kernel(x_ref, pred_ref, lse_ref):
    xb = x_ref[...]
    pred_ref[...] = jnp.argmax(xb, axis=1, keepdims=True).astype(jnp.int32)
    mx = jnp.max(xb, axis=1, keepdims=True)
    lse_ref[...] = mx + jnp.log(jnp.sum(jnp.exp(xb - mx), axis=1, keepdims=True))


_DMAX = 2 * _NMAX


def _seg_kernel(ns_ref, ms_ref, poff_ref, yw_ref, lw_ref, aw_ref,
                lab_ref, lsesum_ref, rm_ref):
    s = pl.program_id(0)
    n0 = ns_ref[2 * s]
    m0 = ms_ref[2 * s]
    n1 = ns_ref[2 * s + 1]
    m1 = ms_ref[2 * s + 1]

    a0 = aw_ref[0, 0]
    a1 = aw_ref[0, 1]
    jpos = (jax.lax.broadcasted_iota(jnp.int32, (8, _W), 0) * _W
            + jax.lax.broadcasted_iota(jnp.int32, (8, _W), 1))
    jpos16 = (jax.lax.broadcasted_iota(jnp.int32, (16, _W), 0) * _W
              + jax.lax.broadcasted_iota(jnp.int32, (16, _W), 1))
    big = jnp.int32(1 << 24)

    def flat_shiftw(v, w, fills):
        frow = jnp.concatenate(
            [jnp.full((1, 1), f, jnp.int32) for f in fills], axis=1)
        left = jnp.concatenate([frow, v[:-1, -w:]], axis=0)
        return jnp.concatenate([left, v[:, :-w]], axis=1)

    a0s = flat_shiftw(a0, 1, [0])
    a1s = flat_shiftw(a1, 1, [0])

    def half_step2(d, u, A, B, bb, r, a_vec, a_sh):
        yb0 = yw_ref[0, u, jnp.clip(d - 2, 0, _NMAX - 1)]
        yb1 = yw_ref[0, u, jnp.clip(d - 1, 0, _NMAX - 1)]
        A1 = flat_shiftw(A, 1, [d - 1])
        A2 = flat_shiftw(A, 2, [big, d - 1])
        B1 = flat_shiftw(B, 1, [big])
        bb1 = flat_shiftw(bb, 1, [yb0])
        bb2 = flat_shiftw(bb, 2, [yb1, yb0])
        sub0 = jnp.where(a_vec == bb1, 0, 1).astype(jnp.int32)
        sub1 = jnp.where(a_vec == bb2, 0, 1).astype(jnp.int32)
        subs = jnp.where(a_sh == bb2, 0, 1).astype(jnp.int32)
        dc0 = B + sub0
        up0 = A1 + 1
        vd = jnp.where(jpos == d - 1, d,
                       jnp.minimum(jnp.minimum(A + 1, up0), dc0))
        jc4 = ((d - 1) - jpos) * 4
        code0 = jnp.where(vd == dc0, jc4, jnp.where(vd == up0, jc4 + 1, 0))
        ra = jnp.maximum(r, code0)
        shvd = jnp.where(jpos == d, d,
                         jnp.minimum(jnp.minimum(A1 + 1, A2 + 1), B1 + subs))
        dc1 = A1 + sub1
        up1 = shvd + 1
        vd1 = jnp.where(jpos == d, d + 1,
                        jnp.minimum(jnp.minimum(vd + 1, up1), dc1))
        jc4b = (d - jpos) * 4
        code1 = jnp.where(vd1 == dc1, jc4b, jnp.where(vd1 == up1, jc4b + 1, 0))
        rb = jnp.maximum(ra, code1)
        return vd1, shvd, bb2, ra, rb

    def step(t, carry):
        v0, sh0, bb0, r0, v1, sh1, bb1, r1 = carry
        d = 2 + 2 * t
        v0, sh0, bb0, r0a, r0 = half_step2(d, 0, v0, sh0, bb0, r0, a0, a0s)
        v1, sh1, bb1, r1a, r1 = half_step2(d, 1, v1, sh1, bb1, r1, a1, a1s)
        rowa = jnp.concatenate([r0a, r1a], axis=0).astype(jnp.int16)
        rowb = jnp.concatenate([r0, r1], axis=0).astype(jnp.int16)
        rm_ref[pl.ds(d - 2, 2)] = jnp.concatenate(
            [rowa[None], rowb[None]], axis=0)
        return (v0, sh0, bb0, r0, v1, sh1, bb1, r1)

    vinit = jnp.where(jpos == 0, 1, big).astype(jnp.int32)
    shinit = jnp.where(jpos == 0, 0, big).astype(jnp.int32)
    zeros = jnp.zeros((8, _W), jnp.int32)
    dmax = jnp.maximum(n0 + m0, n1 + m1)
    jax.lax.fori_loop(0, dmax // 2, step,
                      (vinit, shinit, zeros, zeros,
                       vinit, shinit, zeros, zeros))

    def clr(t, _):
        for q in range(8):
            lab_ref[0, 0, t * 8 + q] = jnp.int32(-1)
            lab_ref[0, 1, t * 8 + q] = jnp.int32(-1)
        return 0

    jax.lax.fori_loop(0, _XW // 8, clr, 0)

    def tb_post(cmax, i, j, acc, u, active):
        p = i - 1
        col = jax.lax.shift_right_logical(cmax, 2)
        found = active & (cmax > 3)
        isdiag = found & ((cmax & 3) == 0)
        lab = yw_ref[0, u, jnp.clip(col - 1, 0, _NMAX - 1)]
        idx = jnp.where(isdiag, poff_ref[2 * s + u] + p, _XW - 1)
        lab_ref[0, u, idx] = jnp.where(isdiag, lab, jnp.int32(-1))
        acc = acc + jnp.where(isdiag, lw_ref[0, u, p], 0.0)
        i2 = jnp.where(found, i - 1, i)
        j2 = jnp.where(isdiag, col - 1,
                       jnp.where(found, col, jnp.where(active, 0, j)))
        return i2, j2, acc

    def tb_cond(st):
        i0, j0, acc0, i1, j1, acc1 = st
        return ((i0 > 0) & (j0 > 0)) | ((i1 > 0) & (j1 > 0))

    def tb_body(st):
        i0, j0, acc0, i1, j1, acc1 = st
        act0 = (i0 > 0) & (j0 > 0)
        act1 = (i1 > 0) & (j1 > 0)
        row0 = rm_ref[jnp.clip(i0 + j0 - 2, 0, _DMAX - 1)][0:8]
        row1 = rm_ref[jnp.clip(i1 + j1 - 2, 0, _DMAX - 1)][8:16]
        comb = (jnp.where(jpos == i0 - 1, row0.astype(jnp.int32), 0)
                + jnp.where(jpos == i1 - 1, row1.astype(jnp.int32), 0) * 65536)
        c = jnp.sum(comb)
        i0, j0, acc0 = tb_post(c & 0xFFFF, i0, j0, acc0, 0, act0)
        i1, j1, acc1 = tb_post(
            jax.lax.shift_right_logical(c, 16), i1, j1, acc1, 1, act1)
        return (i0, j0, acc0, i1, j1, acc1)

    _, _, acc0, _, _, acc1 = jax.lax.while_loop(
        tb_cond, tb_body,
        (n0, m0, jnp.float32(0.0), n1, m1, jnp.float32(0.0)))
    lsesum_ref[0, 0, 0] = acc0
    lsesum_ref[0, 1, 0] = acc1


def _ce_kernel(ns_ref, ms_ref, xs_ref, lsesum_ref, lab_ref, x_ref,
               out_ref, acc_ref):
    s = pl.program_id(0)
    nseg = pl.num_programs(0)

    @pl.when(s == 0)
    def _():
        acc_ref[0] = 0.0
        acc_ref[1] = 0.0

    xstart = pl.multiple_of(xs_ref[s], 8)
    xb = x_ref[pl.ds(xstart, _XW), :]
    lab2 = lab_ref[0]
    lanes = jax.lax.broadcasted_iota(jnp.int32, (_XW, x_ref.shape[1]), 1)
    xdot = jnp.sum(jnp.where(lab2 == lanes, xb, 0.0))
    cnt = jnp.sum(jnp.where(lab2 >= 0, 1, 0))

    nonempty = (ns_ref[s] > 0) & (ms_ref[s] > 0)
    ce_sum = lsesum_ref[s, 0] - xdot
    seg_mean = ce_sum / jnp.maximum(cnt, 1).astype(jnp.float32)
    acc_ref[0] = acc_ref[0] + jnp.where(nonempty, seg_mean, 0.0)
    acc_ref[1] = acc_ref[1] + jnp.where(nonempty, 0.0, 1.0)

    @pl.when(s == nseg - 1)
    def _():
        out_ref[0, 0] = acc_ref[0] / (jnp.float32(nseg) - acc_ref[1])


def kernel(x, y, num_chars, num_labels):
    Lx, C = x.shape
    Ly = y.shape[0]
    S = num_chars.shape[0]

    y32 = y.astype(jnp.int32)
    nc = num_chars.astype(jnp.int32)
    nl = num_labels.astype(jnp.int32)

    BR = min(1024, Lx)
    pred2, lse2 = pl.pallas_call(
        _stats_kernel,
        grid=(Lx // BR,),
        in_specs=[pl.BlockSpec((BR, C), lambda i: (i, 0))],
        out_specs=[pl.BlockSpec((BR, 1), lambda i: (i, 0)),
                   pl.BlockSpec((BR, 1), lambda i: (i, 0))],
        out_shape=[jax.ShapeDtypeStruct((Lx, 1), jnp.int32),
                   jax.ShapeDtypeStruct((Lx, 1), jnp.float32)],
    )(x)
    pred = pred2[:, 0]
    lse = lse2[:, 0]

    pxs, pys, ns, ms = [], [], [], []
    px = jnp.int32(0)
    py = jnp.int32(0)
    for i in range(S):
        n_i = jnp.clip(jnp.minimum(nc[i], Lx - px), 0, _NMAX)
        m_i = jnp.clip(jnp.minimum(nl[i], Ly - py), 0, _NMAX)
        pxs.append(px)
        pys.append(py)
        ns.append(n_i)
        ms.append(m_i)
        ne = (n_i > 0) & (m_i > 0)
        px = px + jnp.where(ne, nc[i], 0)
        py = py + jnp.where(ne, nl[i], 0)
    pxs = jnp.stack(pxs)
    pys = jnp.stack(pys)
    ns = jnp.stack(ns)
    ms = jnp.stack(ms)

    xstarts = jnp.minimum((pxs // 8) * 8, Lx - _XW)
    poff = pxs - xstarts

    k2 = jnp.arange(_NMAX, dtype=jnp.int32)
    gx = jnp.clip(pxs[:, None] + k2[None, :], 0, Lx - 1)
    gy = jnp.clip(pys[:, None] + k2[None, :], 0, Ly - 1)
    aw = pred[gx]
    yw = y32[gy]
    lw = lse[gx]

    lab, lsesum = pl.pallas_call(
        _seg_kernel,
        grid=(S // 2,),
        in_specs=[
            pl.BlockSpec(memory_space=pltpu.SMEM),
            pl.BlockSpec(memory_space=pltpu.SMEM),
            pl.BlockSpec(memory_space=pltpu.SMEM),
            pl.BlockSpec((1, 2, _NMAX), lambda s: (s, 0, 0),
                         memory_space=pltpu.SMEM),
            pl.BlockSpec((1, 2, _NMAX), lambda s: (s, 0, 0),
                         memory_space=pltpu.SMEM),
            pl.BlockSpec((1, 2, 8, _W), lambda s: (s, 0, 0, 0)),
        ],
        out_specs=[
            pl.BlockSpec((1, 2, _XW), lambda s: (s, 0, 0),
                         memory_space=pltpu.SMEM),
            pl.BlockSpec((1, 2, 1), lambda s: (s, 0, 0),
                         memory_space=pltpu.SMEM),
        ],
        out_shape=[jax.ShapeDtypeStruct((S // 2, 2, _XW), jnp.int32),
                   jax.ShapeDtypeStruct((S // 2, 2, 1), jnp.float32)],
        scratch_shapes=[pltpu.VMEM((_DMAX, 16, _W), jnp.int16)],
        compiler_params=_CP(vmem_limit_bytes=48 * 1024 * 1024),
    )(ns, ms, poff, yw.reshape(S // 2, 2, _NMAX), lw.reshape(S // 2, 2, _NMAX),
      aw.reshape(S // 2, 2, 8, _W))

    out = pl.pallas_call(
        _ce_kernel,
        grid=(S,),
        in_specs=[
            pl.BlockSpec(memory_space=pltpu.SMEM),
            pl.BlockSpec(memory_space=pltpu.SMEM),
            pl.BlockSpec(memory_space=pltpu.SMEM),
            pl.BlockSpec(memory_space=pltpu.SMEM),
            pl.BlockSpec((1, _XW, 1), lambda s: (s, 0, 0)),
            pl.BlockSpec((Lx, C), lambda s: (0, 0)),
        ],
        out_specs=pl.BlockSpec(memory_space=pltpu.SMEM),
        out_shape=jax.ShapeDtypeStruct((1, 1), jnp.float32),
        scratch_shapes=[pltpu.SMEM((2,), jnp.float32)],
        compiler_params=_CP(vmem_limit_bytes=40 * 1024 * 1024),
    )(ns, ms, xstarts, lsesum.reshape(S, 1), lab.reshape(S, _XW, 1), x)
    return out[0, 0]
